# baseline (speedup 1.0000x reference)
import functools

import jax
import jax.numpy as jnp
from jax import lax
from jax.experimental import pallas as pl
from jax.experimental.pallas import tpu as pltpu
from jax.experimental.pallas import tpu_sc as plsc

B, L, D = 1024, 200, 128
MAX_POS = 10000
LANE = 16
_info = plsc.get_sparse_core_info()
NC, NS = _info.num_cores, _info.num_subcores
NW = NC * NS
ROWS_PER_W = B // NW
CH = (104, 80, 16)
OFF = (0, 104, 184)
NCH = len(CH)

_mesh = plsc.VectorSubcoreMesh(core_axis_name="c", subcore_axis_name="s")

_GATHER_DNUMS = lax.GatherDimensionNumbers(
    offset_dims=(), collapsed_slice_dims=(0,), start_index_map=(0,))


def _lane_permute(x, perm):
    return lax.gather(
        x, perm[:, None], _GATHER_DNUMS, slice_sizes=(1,),
        mode=lax.GatherScatterMode.PROMISE_IN_BOUNDS)


@functools.partial(
    pl.kernel,
    out_type=jax.ShapeDtypeStruct((B, L, D), jnp.float32),
    mesh=_mesh,
    scratch_types=[
        pltpu.VMEM((ROWS_PER_W, L), jnp.int32),
        pltpu.VMEM((L,), jnp.int32),
        pltpu.VMEM((L,), jnp.int32),
        pltpu.VMEM((CH[0], D), jnp.float32),
        pltpu.VMEM((CH[1], D), jnp.float32),
        pltpu.VMEM((CH[2], D), jnp.float32),
        pltpu.VMEM_SHARED((MAX_POS, D), jnp.float32),
        pltpu.SemaphoreType.DMA,
        pltpu.SemaphoreType.DMA,
        pltpu.SemaphoreType.DMA,
    ],
)
def _pe_kernel(vco_hbm, table_hbm, out_hbm, idx_v, rel0_v, rel1_v,
               buf0, buf1, buf2, table_sh, sem0, sem1, sem2):
    wid = lax.axis_index("s") * NC + lax.axis_index("c")
    base = wid * ROWS_PER_W
    sid = lax.axis_index("s")
    stage_rows = (MAX_POS // NS) // 8 * 8
    tail = MAX_POS - NS * stage_rows
    pltpu.sync_copy(table_hbm.at[pl.ds(sid * stage_rows, stage_rows)],
                    table_sh.at[pl.ds(sid * stage_rows, stage_rows)])

    @pl.when(sid == 0)
    def _stage_tail():
        pltpu.sync_copy(table_hbm.at[pl.ds(NS * stage_rows, tail)],
                        table_sh.at[pl.ds(NS * stage_rows, tail)])

    pltpu.sync_copy(vco_hbm.at[pl.ds(base, ROWS_PER_W)], idx_v)
    plsc.subcore_barrier()

    rels = (rel0_v, rel1_v)
    bufs = (buf0, buf1, buf2)
    sems = (sem0, sem1, sem2)
    srcs = (table_sh, table_sh, table_hbm)

    def compute_rel(r, p):
        m = idx_v[r, pl.ds(0, LANE)]
        for k in range(1, L // LANE):
            m = jnp.minimum(m, idx_v[r, pl.ds(k * LANE, LANE)])
        m = jnp.minimum(m, idx_v[r, pl.ds(L - LANE, LANE)])
        lanes = lax.iota(jnp.int32, LANE)
        for sh in (8, 4, 2, 1):
            perm = lax.rem(lanes + sh, LANE)
            m = jnp.minimum(m, _lane_permute(m, perm))
        rel_v = rels[p]
        for k in range(L // LANE):
            rel_v[pl.ds(k * LANE, LANE)] = jnp.abs(
                idx_v[r, pl.ds(k * LANE, LANE)] - m)
        rel_v[pl.ds(L - LANE, LANE)] = jnp.abs(
            idx_v[r, pl.ds(L - LANE, LANE)] - m)

    def fire_gather(p, h):
        pltpu.async_copy(
            srcs[h].at[rels[p].at[pl.ds(OFF[h], CH[h])]],
            bufs[h], sems[h])

    def wait_gather(p, h):
        pltpu.make_async_copy(
            srcs[h].at[rels[p].at[pl.ds(OFF[h], CH[h])]],
            bufs[h], sems[h]).wait()

    def copy_out(r, h):
        pltpu.sync_copy(bufs[h], out_hbm.at[r + base, pl.ds(OFF[h], CH[h])])

    compute_rel(0, 0)
    for h in range(NCH):
        fire_gather(0, h)

    def step(s, carry):
        r = 2 * s
        compute_rel(r + 1, 1)
        for h in range(NCH):
            wait_gather(0, h)
            copy_out(r, h)
            fire_gather(1, h)
        compute_rel(r + 2, 0)
        for h in range(NCH):
            wait_gather(1, h)
            copy_out(r + 1, h)
            fire_gather(0, h)
        return carry

    lax.fori_loop(0, (ROWS_PER_W - 2) // 2, step, 0)

    last = ROWS_PER_W - 1
    compute_rel(last, 1)
    for h in range(NCH):
        wait_gather(0, h)
        copy_out(last - 1, h)
        fire_gather(1, h)
    for h in range(NCH):
        wait_gather(1, h)
        copy_out(last, h)


def kernel(visit_concept_orders, pos_encoding):
    return _pe_kernel(visit_concept_orders.astype(jnp.int32), pos_encoding)
